# baseline (speedup 1.0000x reference)
import jax
import jax.numpy as jnp
from jax import lax
from jax.experimental import pallas as pl
from jax.experimental.pallas import tpu as pltpu
from jax.experimental.pallas import tpu_sc as plsc

NC = 2
NS = 16
NW = NC * NS
L = 200
D = 64
B = 4096
V = 1000000
BPW = B // NW
NBUF = 4
LA = 2
NG = L // NBUF


def _emb_body(ids_hbm, table_hbm, pos_hbm, out_hbm,
              idx_v, pos_v, b0, b1, b2, b3,
              g0, g1, g2, g3, o0, o1, o2, o3):
    wid = lax.axis_index("s") * NC + lax.axis_index("c")
    row0 = wid * BPW
    pltpu.sync_copy(ids_hbm.at[:, pl.ds(row0, BPW)], idx_v)
    pltpu.sync_copy(pos_hbm, pos_v)

    bufs = (b0, b1, b2, b3)
    gsems = (g0, g1, g2, g3)
    osems = (o0, o1, o2, o3)

    def gather(l, b):
        return pltpu.make_async_copy(table_hbm.at[idx_v.at[l]], bufs[b], gsems[b])

    def outcopy(l, b):
        return pltpu.make_async_copy(
            bufs[b], out_hbm.at[l, pl.ds(row0, BPW), :], osems[b])

    gather(0, 0).start()
    gather(1, 1).start()

    @pl.loop(0, NG)
    def group(g):
        for b in range(NBUF):
            l = NBUF * g + b
            buf = bufs[b]
            gather(l, b).wait()
            pv = [pos_v[pl.ds(l * D + q * 16, 16)] for q in range(D // 16)]
            for k in range(BPW):
                for q in range(D // 16):
                    plsc.addupdate(buf.at[k, pl.ds(q * 16, 16)], pv[q])
            outcopy(l, b).start()
            bn = (b + LA) % NBUF
            ln = l + LA
            if b < LA:
                @pl.when(g >= 1)
                def _wait():
                    outcopy(ln - NBUF, bn).wait()
                gather(ln, bn).start()
            else:
                outcopy(ln - NBUF, bn).wait()

                @pl.when(g < NG - 1)
                def _go():
                    gather(ln, bn).start()

    for b in range(LA, NBUF):
        outcopy(NBUF * (NG - 1) + b, b).wait()


def kernel(input_ids, token_table, pos_table):
    mesh = plsc.VectorSubcoreMesh(core_axis_name="c", subcore_axis_name="s")
    f = pl.kernel(
        _emb_body,
        out_type=jax.ShapeDtypeStruct((L, B, D), jnp.float32),
        mesh=mesh,
        scratch_types=[
            pltpu.VMEM((L, BPW), jnp.int32),
            pltpu.VMEM((L * D,), jnp.float32),
            pltpu.VMEM((BPW, D), jnp.float32),
            pltpu.VMEM((BPW, D), jnp.float32),
            pltpu.VMEM((BPW, D), jnp.float32),
            pltpu.VMEM((BPW, D), jnp.float32),
        ] + [pltpu.SemaphoreType.DMA] * 8,
        compiler_params=pltpu.CompilerParams(use_tc_tiling_on_sc=False),
    )
    ids_t = input_ids.astype(jnp.int32).T
    pos_flat = pos_table[:L].reshape(L * D)
    out = f(ids_t, token_table, pos_flat)
    return jnp.transpose(out, (1, 0, 2))
